# baseline (speedup 1.0000x reference)
import dataclasses
import functools
import math

import jax
import jax.numpy as jnp
from jax import lax
from jax.experimental import pallas as pl
from jax.experimental.pallas import tpu as pltpu
from jax.experimental.pallas import tpu_sc as plsc

N = 320000
D = 128
OUTF = 128
NSEG = 64000
NC = 2
NS = 16
NW = NC * NS
SEG_PER_W = NSEG // NW
CHUNK = 80
NCHUNK = SEG_PER_W // CHUNK
RB = 128
NBLK = N // RB
NL = 16
NFC = D // NL


def _sc_stats(x, segment_ids):
    mesh = plsc.VectorSubcoreMesh(core_axis_name="c", subcore_axis_name="s")
    stat = jax.ShapeDtypeStruct((NSEG, D), jnp.float32)
    cp = pltpu.CompilerParams()
    if "needs_layout_passes" in pltpu.CompilerParams.__dataclass_fields__:
        cp = dataclasses.replace(cp, needs_layout_passes=False)
    kernel_fn = pl.kernel(
        _sc_body,
        out_type=(stat, stat, stat, stat, stat),
        mesh=mesh,
        compiler_params=cp,
        scratch_types=[
            pltpu.VMEM((RB, D), jnp.float32),
            pltpu.VMEM((RB, D), jnp.float32),
            pltpu.VMEM((RB,), jnp.int32),
            pltpu.VMEM((RB,), jnp.int32),
            pltpu.VMEM((16,), jnp.int32),
            pltpu.VMEM((CHUNK + 8, D), jnp.float32),
            pltpu.VMEM((CHUNK + 8, D), jnp.float32),
            pltpu.VMEM((CHUNK + 8, D), jnp.float32),
            pltpu.VMEM((CHUNK + 8, D), jnp.float32),
            pltpu.VMEM((CHUNK + 8, D), jnp.float32),
            pltpu.SMEM((1,), jnp.int32),
            pltpu.SemaphoreType.DMA,
            pltpu.SemaphoreType.DMA,
            pltpu.SemaphoreType.DMA,
        ],
    )
    return kernel_fn(x, segment_ids)


def _sc_body(x_hbm, ids_hbm, o_sum, o_sq, o_mn, o_mx, o_cnt,
             xv0, xv1, iv0, iv1, probe,
             st_sum, st_sq, st_mn, st_mx, st_cnt,
             ckr, s0, s1, sf):
    wid = lax.axis_index("s") * NC + lax.axis_index("c")
    seg_lo = (wid * SEG_PER_W).astype(jnp.int32)

    zvec = jnp.zeros((NL,), jnp.float32)

    def cnt_reset_body(slot, carry):
        for c in range(NFC):
            st_cnt[slot, pl.ds(c * NL, NL)] = zvec
        return carry

    lax.fori_loop(0, CHUNK, cnt_reset_body, 0)
    ckr[0] = jnp.int32(0)

    def lower_bound(target):
        nb16 = N // 16

        def body(_, state):
            lo, hi = state
            active = lo < hi
            mid = lax.div(lo + hi, 2)
            mid_c = jnp.minimum(mid, nb16 - 1)
            off = pl.multiple_of(mid_c * 16, 16)
            pltpu.sync_copy(ids_hbm.at[pl.ds(off, 16)], probe)
            v0 = probe[...][0]
            take = v0 < target
            lo = jnp.where(active & take, mid + 1, lo)
            hi = jnp.where(active & (~take), mid, hi)
            return lo, hi

        blk, _ = lax.fori_loop(0, 15, body,
                               (jnp.int32(0), jnp.int32(nb16)))
        ws = pl.multiple_of(jnp.maximum(blk * 16 - 16, 0), 16)
        pltpu.sync_copy(ids_hbm.at[pl.ds(ws, 16)], probe)
        cnt = plsc.all_reduce_population_count(probe[...] < target)
        return ws + cnt[0]

    rs = lower_bound(seg_lo)
    re = lower_bound(seg_lo + SEG_PER_W)

    def flush_chunk(k, carry):
        base = seg_lo + k * CHUNK
        dst = pl.ds(base, CHUNK)
        src = pl.ds(0, CHUNK)
        hs = [
            pltpu.async_copy(st_sum.at[src], o_sum.at[dst], sf),
            pltpu.async_copy(st_sq.at[src], o_sq.at[dst], sf),
            pltpu.async_copy(st_mn.at[src], o_mn.at[dst], sf),
            pltpu.async_copy(st_mx.at[src], o_mx.at[dst], sf),
            pltpu.async_copy(st_cnt.at[src], o_cnt.at[dst], sf),
        ]
        for h in hs:
            h.wait()
        lax.fori_loop(0, CHUNK, cnt_reset_body, 0)
        return carry

    def store_run(cs, cnt, accs):
        t = lax.div(cs - seg_lo, jnp.int32(CHUNK))
        lax.fori_loop(ckr[0], t, flush_chunk, 0)
        ckr[0] = t
        slot = cs - seg_lo - t * CHUNK
        cv = jnp.broadcast_to(cnt, (NL,))
        for c in range(NFC):
            sl = pl.ds(c * NL, NL)
            st_sum[slot, sl] = accs[c]
            st_sq[slot, sl] = accs[NFC + c]
            st_mn[slot, sl] = accs[2 * NFC + c]
            st_mx[slot, sl] = accs[3 * NFC + c]
            st_cnt[slot, sl] = cv

    def issue(j, xv, iv, sem):
        roff = pl.multiple_of(j * RB, RB)
        pltpu.async_copy(x_hbm.at[pl.ds(roff, RB)], xv, sem)
        pltpu.async_copy(ids_hbm.at[pl.ds(roff, RB)], iv, sem)

    def wait_bufs(xv, iv, sem):
        pltpu.make_async_copy(x_hbm.at[pl.ds(0, RB)], xv, sem).wait()
        pltpu.make_async_copy(ids_hbm.at[pl.ds(0, RB)], iv, sem).wait()

    def process_block(j, xv, iv, state):
        row0 = j * RB
        i0 = jnp.maximum(rs - row0, 0)
        i1 = jnp.maximum(jnp.minimum(re - row0, RB), i0)

        def group_body(g, state):
            g16 = pl.multiple_of(g * 16, 16)
            idvec = iv[pl.ds(g16, 16)]
            for l in range(16):
                cs, cnt = state[0], state[1]
                accs = state[2:]
                r = g16 + l
                s = idvec[l]
                ok = (r >= i0) & (r < i1)
                s_eff = jnp.where(ok, s, jnp.int32(-2))
                b = s_eff != cs
                flushable = b & (cs >= 0)

                @pl.when(flushable)
                def _():
                    store_run(cs, cnt, accs)

                cs = jnp.where(b, s_eff, cs)
                inc = jnp.where(ok, 1.0, 0.0)
                cnt = jnp.where(b, inc, cnt + inc)
                nsum, nsq, nmn, nmx = [], [], [], []
                for c in range(NFC):
                    sl = pl.ds(c * NL, NL)
                    v = xv[r, sl]
                    nsum.append(jnp.where(b, 0.0, accs[c]) + v)
                    nsq.append(jnp.where(b, 0.0, accs[NFC + c]) + v * v)
                    nmn.append(jnp.minimum(
                        jnp.where(b, jnp.inf, accs[2 * NFC + c]), v))
                    nmx.append(jnp.maximum(
                        jnp.where(b, -jnp.inf, accs[3 * NFC + c]), v))
                state = (cs, cnt) + tuple(nsum + nsq + nmn + nmx)
            return state

        return lax.fori_loop(0, RB // 16, group_body, state)

    jb0 = lax.div(rs, jnp.int32(RB))
    jbe = lax.div(re + (RB - 1), jnp.int32(RB))
    jbe = jnp.maximum(jbe, jb0)
    nblk = jbe - jb0

    @pl.when(nblk > 0)
    def _():
        issue(jb0, xv0, iv0, s0)

    def pair_body(it, state):
        j0 = jb0 + 2 * it
        j1 = j0 + 1
        wait_bufs(xv0, iv0, s0)

        @pl.when(j1 < jbe)
        def _():
            issue(j1, xv1, iv1, s1)

        state = process_block(j0, xv0, iv0, state)

        def do_second(state):
            wait_bufs(xv1, iv1, s1)

            @pl.when(j1 + 1 < jbe)
            def _():
                issue(j1 + 1, xv0, iv0, s0)

            return process_block(j1, xv1, iv1, state)

        return lax.cond(j1 < jbe, do_second, lambda st: st, state)

    state0 = (jnp.int32(-1), jnp.float32(0.0)) + tuple(
        zvec for _ in range(4 * NFC))
    npairs = lax.div(nblk + 1, jnp.int32(2))
    state = lax.fori_loop(0, npairs, pair_body, state0)

    cs, cnt = state[0], state[1]
    accs = state[2:]

    @pl.when(cs >= 0)
    def _():
        store_run(cs, cnt, accs)

    lax.fori_loop(ckr[0], NCHUNK, flush_chunk, 0)


def _combine(sums, sqs, mns, mxs, cnts, W, b2):
    BS = 512
    c5 = 1.0 / math.sqrt(5.0)

    def body(su, sq, mn, mx, cn, w_ref, b_ref, o_ref):
        cnt = cn[...]
        has = cnt > 0.0
        inv = 1.0 / jnp.maximum(cnt, 1.0)
        s = jnp.where(has, su[...], 0.0)
        mean = s * inv
        msq = jnp.where(has, sq[...], 0.0) * inv
        std = jnp.sqrt(jnp.maximum(msq - mean * mean, 1e-9))
        mn0 = jnp.where(has, mn[...], 0.0)
        mx0 = jnp.where(has, mx[...], 0.0)
        w = w_ref[...]
        dot = functools.partial(jnp.dot, precision=lax.Precision.HIGHEST,
                                preferred_element_type=jnp.float32)
        acc = dot(s * c5, w[0:128])
        acc += dot(mean, w[128:256])
        acc += dot(mn0, w[256:384])
        acc += dot(mx0, w[384:512])
        acc += dot(std, w[512:640])
        o_ref[...] = acc + b_ref[...]

    blk = lambda i: (i, 0)
    zero = lambda i: (0, 0)
    return pl.pallas_call(
        body,
        grid=(NSEG // BS,),
        in_specs=[pl.BlockSpec((BS, D), blk)] * 5
        + [pl.BlockSpec((5 * D, OUTF), zero), pl.BlockSpec((1, OUTF), zero)],
        out_specs=pl.BlockSpec((BS, OUTF), blk),
        out_shape=jax.ShapeDtypeStruct((NSEG, OUTF), jnp.float32),
    )(sums, sqs, mns, mxs, cnts, W, b2)


def kernel(x, segment_ids, W, b):
    sums, sqs, mns, mxs, cnts = _sc_stats(x, segment_ids.astype(jnp.int32))
    return _combine(sums, sqs, mns, mxs, cnts, W, b.reshape(1, OUTF))


# speedup vs baseline: 3.5049x; 3.5049x over previous
import dataclasses
import functools
import math

import jax
import jax.numpy as jnp
from jax import lax
from jax.experimental import pallas as pl
from jax.experimental.pallas import tpu as pltpu
from jax.experimental.pallas import tpu_sc as plsc

N = 320000
D = 128
OUTF = 128
NSEG = 64000
NC = 2
NS = 16
NW = NC * NS
SEG_PER_W = NSEG // NW
CHUNK = 80
NCHUNK = SEG_PER_W // CHUNK
RB = 128
NBLK = N // RB
NL = 16
NFC = D // NL


def _sc_stats(x, segment_ids):
    mesh = plsc.VectorSubcoreMesh(core_axis_name="c", subcore_axis_name="s")
    stat = jax.ShapeDtypeStruct((NSEG, D), jnp.float32)
    cp = pltpu.CompilerParams()
    if "needs_layout_passes" in pltpu.CompilerParams.__dataclass_fields__:
        cp = dataclasses.replace(cp, needs_layout_passes=False)
    kernel_fn = pl.kernel(
        _sc_body,
        out_type=(stat, stat, stat, stat, stat),
        mesh=mesh,
        compiler_params=cp,
        scratch_types=[
            pltpu.VMEM((RB, D), jnp.float32),
            pltpu.VMEM((RB, D), jnp.float32),
            pltpu.VMEM((RB,), jnp.int32),
            pltpu.VMEM((RB,), jnp.int32),
            pltpu.VMEM((16,), jnp.int32),
            pltpu.VMEM((CHUNK + 8, D), jnp.float32),
            pltpu.VMEM((CHUNK + 8, D), jnp.float32),
            pltpu.VMEM((CHUNK + 8, D), jnp.float32),
            pltpu.VMEM((CHUNK + 8, D), jnp.float32),
            pltpu.VMEM((CHUNK + 8, D), jnp.float32),
            pltpu.SMEM((1,), jnp.int32),
            pltpu.SemaphoreType.DMA,
            pltpu.SemaphoreType.DMA,
            pltpu.SemaphoreType.DMA,
        ],
    )
    return kernel_fn(x, segment_ids)


def _sc_body(x_hbm, ids_hbm, o_sum, o_sq, o_mn, o_mx, o_cnt,
             xv0, xv1, iv0, iv1, probe,
             st_sum, st_sq, st_mn, st_mx, st_cnt,
             ckr, s0, s1, sf):
    wid = lax.axis_index("s") * NC + lax.axis_index("c")
    seg_lo = (wid * SEG_PER_W).astype(jnp.int32)

    zvec = jnp.zeros((NL,), jnp.float32)

    def cnt_reset_body(slot, carry):
        for c in range(NFC):
            st_cnt[slot, pl.ds(c * NL, NL)] = zvec
        return carry

    lax.fori_loop(0, CHUNK, cnt_reset_body, 0)
    ckr[0] = jnp.int32(0)

    def lower_bound(target):
        nb16 = N // 16

        def body(_, state):
            lo, hi = state
            active = lo < hi
            mid = lax.div(lo + hi, 2)
            mid_c = jnp.minimum(mid, nb16 - 1)
            off = pl.multiple_of(mid_c * 16, 16)
            pltpu.sync_copy(ids_hbm.at[pl.ds(off, 16)], probe)
            v0 = probe[...][0]
            take = v0 < target
            lo = jnp.where(active & take, mid + 1, lo)
            hi = jnp.where(active & (~take), mid, hi)
            return lo, hi

        blk, _ = lax.fori_loop(0, 15, body,
                               (jnp.int32(0), jnp.int32(nb16)))
        ws = pl.multiple_of(jnp.maximum(blk * 16 - 16, 0), 16)
        pltpu.sync_copy(ids_hbm.at[pl.ds(ws, 16)], probe)
        cnt = plsc.all_reduce_population_count(probe[...] < target)
        return ws + cnt[0]

    rs = lower_bound(seg_lo)
    re = lower_bound(seg_lo + SEG_PER_W)

    def flush_chunk(k, carry):
        base = seg_lo + k * CHUNK
        dst = pl.ds(base, CHUNK)
        src = pl.ds(0, CHUNK)
        hs = [
            pltpu.async_copy(st_sum.at[src], o_sum.at[dst], sf),
            pltpu.async_copy(st_sq.at[src], o_sq.at[dst], sf),
            pltpu.async_copy(st_mn.at[src], o_mn.at[dst], sf),
            pltpu.async_copy(st_mx.at[src], o_mx.at[dst], sf),
            pltpu.async_copy(st_cnt.at[src], o_cnt.at[dst], sf),
        ]
        for h in hs:
            h.wait()
        lax.fori_loop(0, CHUNK, cnt_reset_body, 0)
        return carry

    def store_run(cs, cnt, accs):
        t = lax.div(cs - seg_lo, jnp.int32(CHUNK))
        lax.fori_loop(ckr[0], t, flush_chunk, 0)
        ckr[0] = t
        slot = cs - seg_lo - t * CHUNK
        cv = jnp.broadcast_to(cnt, (NL,))
        for c in range(NFC):
            sl = pl.ds(c * NL, NL)
            st_sum[slot, sl] = accs[c]
            st_sq[slot, sl] = accs[NFC + c]
            st_mn[slot, sl] = accs[2 * NFC + c]
            st_mx[slot, sl] = accs[3 * NFC + c]
            st_cnt[slot, sl] = cv

    def store_run_at(adv, cs, cnt, accs):
        slot = cs - seg_lo - adv * CHUNK
        cv = jnp.broadcast_to(cnt, (NL,))
        for c in range(NFC):
            sl = pl.ds(c * NL, NL)
            st_sum[slot, sl] = accs[c]
            st_sq[slot, sl] = accs[NFC + c]
            st_mn[slot, sl] = accs[2 * NFC + c]
            st_mx[slot, sl] = accs[3 * NFC + c]
            st_cnt[slot, sl] = cv

    def issue(j, xv, iv, sem):
        roff = pl.multiple_of(j * RB, RB)
        pltpu.async_copy(x_hbm.at[pl.ds(roff, RB)], xv, sem)
        pltpu.async_copy(ids_hbm.at[pl.ds(roff, RB)], iv, sem)

    def wait_bufs(xv, iv, sem):
        pltpu.make_async_copy(x_hbm.at[pl.ds(0, RB)], xv, sem).wait()
        pltpu.make_async_copy(ids_hbm.at[pl.ds(0, RB)], iv, sem).wait()

    def process_block(j, xv, iv, state):
        row0 = j * RB
        i0 = jnp.maximum(rs - row0, 0)
        i1 = jnp.maximum(jnp.minimum(re - row0, RB), i0)

        def lane_core(l, g16, idvec, state, close_fn):
            cs, cnt = state[0], state[1]
            accs = state[2:]
            r = g16 + l
            s = idvec[l]
            ok = (r >= i0) & (r < i1)
            s_eff = jnp.where(ok, s, jnp.int32(-2))
            b = s_eff != cs
            flushable = b & (cs >= 0)

            @pl.when(flushable)
            def _():
                close_fn(cs, cnt, accs)

            cs = jnp.where(b, s_eff, cs)
            inc = jnp.where(ok, 1.0, 0.0)
            cnt = jnp.where(b, inc, cnt + inc)
            nsum, nsq, nmn, nmx = [], [], [], []
            for c in range(NFC):
                sl = pl.ds(c * NL, NL)
                v = xv[r, sl]
                nsum.append(jnp.where(b, 0.0, accs[c]) + v)
                nsq.append(jnp.where(b, 0.0, accs[NFC + c]) + v * v)
                nmn.append(jnp.minimum(
                    jnp.where(b, jnp.inf, accs[2 * NFC + c]), v))
                nmx.append(jnp.maximum(
                    jnp.where(b, -jnp.inf, accs[3 * NFC + c]), v))
            return (cs, cnt) + tuple(nsum + nsq + nmn + nmx)

        def group_body(g, state):
            g16 = pl.multiple_of(g * 16, 16)
            idvec = iv[pl.ds(g16, 16)]
            cs = state[0]
            ck0 = ckr[0]
            sgmax = seg_lo + (SEG_PER_W - 1)
            c0 = lax.div(jnp.clip(idvec[0], seg_lo, sgmax) - seg_lo,
                         jnp.int32(CHUNK))
            cF = lax.div(jnp.clip(idvec[15], seg_lo, sgmax) - seg_lo,
                         jnp.int32(CHUNK))
            t_open = jnp.where(cs >= 0,
                               lax.div(cs - seg_lo, jnp.int32(CHUNK)),
                               jnp.int32(NCHUNK))
            adv = jnp.clip(jnp.minimum(t_open, c0), ck0, NCHUNK - 1)
            lax.fori_loop(ck0, adv, flush_chunk, 0)
            ckr[0] = adv
            fast = (c0 == adv) & (cF == adv) & ((cs < 0) | (t_open == adv))

            def fast_fn(st):
                fc = functools.partial(store_run_at, adv)
                for l in range(16):
                    st = lane_core(l, g16, idvec, st, fc)
                return st

            def slow_fn(st):
                for l in range(16):
                    st = lane_core(l, g16, idvec, st, store_run)
                return st

            return lax.cond(fast, fast_fn, slow_fn, state)

        return lax.fori_loop(0, RB // 16, group_body, state)

    jb0 = lax.div(rs, jnp.int32(RB))
    jbe = lax.div(re + (RB - 1), jnp.int32(RB))
    jbe = jnp.maximum(jbe, jb0)
    nblk = jbe - jb0

    def block_body(bi, state):
        j = jb0 + bi
        issue(j, xv0, iv0, s0)
        wait_bufs(xv0, iv0, s0)
        return process_block(j, xv0, iv0, state)

    state0 = (jnp.int32(-1), jnp.float32(0.0)) + tuple(
        zvec for _ in range(4 * NFC))
    state = lax.fori_loop(0, nblk, block_body, state0)

    cs, cnt = state[0], state[1]
    accs = state[2:]

    @pl.when(cs >= 0)
    def _():
        store_run(cs, cnt, accs)

    lax.fori_loop(ckr[0], NCHUNK, flush_chunk, 0)


def _combine(sums, sqs, mns, mxs, cnts, W, b2):
    BS = 512
    c5 = 1.0 / math.sqrt(5.0)

    def body(su, sq, mn, mx, cn, w_ref, b_ref, o_ref):
        cnt = cn[...]
        has = cnt > 0.0
        inv = 1.0 / jnp.maximum(cnt, 1.0)
        s = jnp.where(has, su[...], 0.0)
        mean = s * inv
        msq = jnp.where(has, sq[...], 0.0) * inv
        std = jnp.sqrt(jnp.maximum(msq - mean * mean, 1e-9))
        mn0 = jnp.where(has, mn[...], 0.0)
        mx0 = jnp.where(has, mx[...], 0.0)
        w = w_ref[...]
        dot = functools.partial(jnp.dot, precision=lax.Precision.HIGHEST,
                                preferred_element_type=jnp.float32)
        acc = dot(s * c5, w[0:128])
        acc += dot(mean, w[128:256])
        acc += dot(mn0, w[256:384])
        acc += dot(mx0, w[384:512])
        acc += dot(std, w[512:640])
        o_ref[...] = acc + b_ref[...]

    blk = lambda i: (i, 0)
    zero = lambda i: (0, 0)
    return pl.pallas_call(
        body,
        grid=(NSEG // BS,),
        in_specs=[pl.BlockSpec((BS, D), blk)] * 5
        + [pl.BlockSpec((5 * D, OUTF), zero), pl.BlockSpec((1, OUTF), zero)],
        out_specs=pl.BlockSpec((BS, OUTF), blk),
        out_shape=jax.ShapeDtypeStruct((NSEG, OUTF), jnp.float32),
    )(sums, sqs, mns, mxs, cnts, W, b2)


def kernel(x, segment_ids, W, b):
    sums, sqs, mns, mxs, cnts = _sc_stats(x, segment_ids.astype(jnp.int32))
    return _combine(sums, sqs, mns, mxs, cnts, W, b.reshape(1, OUTF))
